# Initial kernel scaffold; baseline (speedup 1.0000x reference)
import functools

import jax
import jax.numpy as jnp
from jax import lax
from jax.experimental import pallas as pl
from jax.experimental.pallas import tpu as pltpu
from jax.experimental.pallas import tpu_sc as plsc

_NC = 2
_NS = 16
_L = 16


def _sc_fused_out(x2, lens, r2, posf, B, S, D, M):
  NW = _NC * _NS
  rows_per_w = B // NW
  xw = S * D
  lw = M * D
  ow = xw + lw
  n_chunks = lw // _L

  mesh = plsc.VectorSubcoreMesh(core_axis_name="c", subcore_axis_name="s")

  @functools.partial(
      pl.kernel,
      mesh=mesh,
      out_type=jax.ShapeDtypeStruct((B, ow), jnp.float32),
      scratch_types=[
          pltpu.VMEM((2 * lw,), jnp.float32),
          pltpu.VMEM((lw,), jnp.float32),
          pltpu.VMEM((rows_per_w,), jnp.int32),
          pltpu.VMEM((ow,), jnp.float32),
      ],
  )
  def k(x_hbm, len_hbm, r2_hbm, pos_hbm, out_hbm, r2_v, pos_v, len_v, rowbuf):
    wid = lax.axis_index("s") * _NC + lax.axis_index("c")
    base = wid * rows_per_w
    pltpu.sync_copy(r2_hbm, r2_v)
    pltpu.sync_copy(pos_hbm, pos_v)
    pltpu.sync_copy(len_hbm.at[pl.ds(base, rows_per_w)], len_v)
    lane = lax.broadcasted_iota(jnp.int32, (_L,), 0)

    def row_body(r, _):
      grp = (r // _L) * _L
      lchunk = len_v[pl.ds(grp, _L)]
      L_b = jnp.sum(jnp.where(lane == (r - grp), lchunk, 0))
      c = (M - L_b) * D
      b = base + r
      pltpu.sync_copy(x_hbm.at[b], rowbuf.at[pl.ds(0, xw)])

      def chunk_body(j, _):
        o = j * _L
        rowbuf[pl.ds(xw + o, _L)] = (
            r2_v[pl.ds(c + o, _L)] + pos_v[pl.ds(o, _L)]
        )
        return 0

      lax.fori_loop(0, n_chunks, chunk_body, 0, unroll=8)
      pltpu.sync_copy(rowbuf, out_hbm.at[b])
      return 0

    lax.fori_loop(0, rows_per_w, row_body, 0)

  return k(x2, lens, r2, posf)


def _tc_mask(lens2, pad250, B, S, M):
  W = S + M
  BB = 256
  grid = (B // BB,)

  def body(len_ref, pad_ref, out_ref):
    L = len_ref[...]
    col = lax.broadcasted_iota(jnp.int32, (BB, W), 1)
    lc = jnp.where(col - S >= L, -jnp.inf, 0.0).astype(jnp.float32)
    out_ref[...] = jnp.where(col < S, pad_ref[...], lc)

  return pl.pallas_call(
      body,
      grid=grid,
      in_specs=[
          pl.BlockSpec((BB, 1), lambda i: (i, 0)),
          pl.BlockSpec((BB, W), lambda i: (i, 0)),
      ],
      out_specs=pl.BlockSpec((BB, W), lambda i: (i, 0)),
      out_shape=jax.ShapeDtypeStruct((B, W), jnp.float32),
  )(lens2, pad250)


def kernel(x, padding_mask, target_length, rev_table, pos_table):
  B, S, D = x.shape
  M = rev_table.shape[0]
  lens = target_length.astype(jnp.int32)
  rev = rev_table[::-1]
  r2 = jnp.concatenate([rev, rev], axis=0).reshape(2 * M * D)
  posf = pos_table.reshape(M * D)
  x2 = x.reshape(B, S * D)

  out2 = _sc_fused_out(x2, lens, r2, posf, B, S, D, M)
  pad250 = jnp.pad(padding_mask.astype(jnp.float32), ((0, 0), (0, M)))
  mask_out = _tc_mask(lens.reshape(B, 1), pad250, B, S, M)
  return out2.reshape(B, S + M, D), mask_out.astype(x.dtype)


# speedup vs baseline: 3.0687x; 3.0687x over previous
import functools

import jax
import jax.numpy as jnp
from jax import lax
from jax.experimental import pallas as pl
from jax.experimental.pallas import tpu as pltpu
from jax.experimental.pallas import tpu_sc as plsc

_NC = 2
_NS = 16
_L = 16


def _sc_fused_out(x2, lens, r2, posf, B, S, D, M):
  NW = _NC * _NS
  rows_per_w = B // NW
  xw = S * D
  lw = M * D
  ow = xw + lw
  n_chunks = lw // _L

  mesh = plsc.VectorSubcoreMesh(core_axis_name="c", subcore_axis_name="s")

  @functools.partial(
      pl.kernel,
      mesh=mesh,
      out_type=jax.ShapeDtypeStruct((B, ow), jnp.float32),
      scratch_types=[
          pltpu.VMEM((2 * lw,), jnp.float32),
          pltpu.VMEM((lw,), jnp.float32),
          pltpu.VMEM((rows_per_w + _L,), jnp.int32),
          pltpu.VMEM((ow,), jnp.float32),
      ],
  )
  def k(x_hbm, len_hbm, r2_hbm, pos_hbm, out_hbm, r2_v, pos_v, len_v, rowbuf):
    wid = lax.axis_index("s") * _NC + lax.axis_index("c")
    base = wid * rows_per_w
    pltpu.sync_copy(r2_hbm, r2_v)
    pltpu.sync_copy(pos_hbm, pos_v)
    pltpu.sync_copy(len_hbm.at[pl.ds(base, rows_per_w)],
                    len_v.at[pl.ds(0, rows_per_w)])

    def row_body(r, _):
      L_b = len_v[pl.ds(r, _L)][0]
      c = (M - L_b) * D
      b = base + r
      pltpu.sync_copy(x_hbm.at[b], rowbuf.at[pl.ds(0, xw)])

      def chunk_body(j, _):
        o = j * _L
        rowbuf[pl.ds(xw + o, _L)] = (
            r2_v[pl.ds(c + o, _L)] + pos_v[pl.ds(o, _L)]
        )
        return 0

      lax.fori_loop(0, n_chunks, chunk_body, 0, unroll=8)
      pltpu.sync_copy(rowbuf, out_hbm.at[b])
      return 0

    lax.fori_loop(0, rows_per_w, row_body, 0)

  return k(x2, lens, r2, posf)


def _tc_mask(lens2, pad250, B, S, M):
  W = S + M
  BB = 256
  grid = (B // BB,)

  def body(len_ref, pad_ref, out_ref):
    L = len_ref[...]
    col = lax.broadcasted_iota(jnp.int32, (BB, W), 1)
    lc = jnp.where(col - S >= L, -jnp.inf, 0.0).astype(jnp.float32)
    out_ref[...] = jnp.where(col < S, pad_ref[...], lc)

  return pl.pallas_call(
      body,
      grid=grid,
      in_specs=[
          pl.BlockSpec((BB, 1), lambda i: (i, 0)),
          pl.BlockSpec((BB, W), lambda i: (i, 0)),
      ],
      out_specs=pl.BlockSpec((BB, W), lambda i: (i, 0)),
      out_shape=jax.ShapeDtypeStruct((B, W), jnp.float32),
  )(lens2, pad250)


def kernel(x, padding_mask, target_length, rev_table, pos_table):
  B, S, D = x.shape
  M = rev_table.shape[0]
  lens = target_length.astype(jnp.int32)
  rev = rev_table[::-1]
  r2 = jnp.concatenate([rev, rev], axis=0).reshape(2 * M * D)
  posf = pos_table.reshape(M * D)
  x2 = x.reshape(B, S * D)

  out2 = _sc_fused_out(x2, lens, r2, posf, B, S, D, M)
  pad250 = jnp.pad(padding_mask.astype(jnp.float32), ((0, 0), (0, M)))
  mask_out = _tc_mask(lens.reshape(B, 1), pad250, B, S, M)
  return out2.reshape(B, S + M, D), mask_out.astype(x.dtype)


# speedup vs baseline: 3.6148x; 1.1780x over previous
import functools

import jax
import jax.numpy as jnp
from jax import lax
from jax.experimental import pallas as pl
from jax.experimental.pallas import tpu as pltpu
from jax.experimental.pallas import tpu_sc as plsc

_NC = 2
_NS = 16
_L = 16


def _sc_fused_out(x2, lens, r2, posf, B, S, D, M):
  NW = _NC * _NS
  rows_per_w = B // NW
  xw = S * D
  lw = M * D
  ow = xw + lw
  n_chunks = lw // _L

  mesh = plsc.VectorSubcoreMesh(core_axis_name="c", subcore_axis_name="s")

  @functools.partial(
      pl.kernel,
      mesh=mesh,
      out_type=jax.ShapeDtypeStruct((B, ow), jnp.float32),
      scratch_types=[
          pltpu.VMEM((2 * lw,), jnp.float32),
          pltpu.VMEM((lw,), jnp.float32),
          pltpu.VMEM((rows_per_w + _L,), jnp.int32),
          pltpu.VMEM((ow,), jnp.float32),
          pltpu.VMEM((ow,), jnp.float32),
          pltpu.SemaphoreType.DMA,
          pltpu.SemaphoreType.DMA,
          pltpu.SemaphoreType.DMA,
          pltpu.SemaphoreType.DMA,
      ],
  )
  def k(x_hbm, len_hbm, r2_hbm, pos_hbm, out_hbm,
        r2_v, pos_v, len_v, buf0, buf1, sx0, sx1, so0, so1):
    wid = lax.axis_index("s") * _NC + lax.axis_index("c")
    base = wid * rows_per_w
    pltpu.sync_copy(r2_hbm, r2_v)
    pltpu.sync_copy(pos_hbm, pos_v)
    pltpu.sync_copy(len_hbm.at[pl.ds(base, rows_per_w)],
                    len_v.at[pl.ds(0, rows_per_w)])

    def compute_lc(buf, r):
      L_b = len_v[pl.ds(r, _L)][0]
      c = (M - L_b) * D

      def chunk_body(j, _):
        o = j * _L
        buf[pl.ds(xw + o, _L)] = r2_v[pl.ds(c + o, _L)] + pos_v[pl.ds(o, _L)]
        return 0

      lax.fori_loop(0, n_chunks, chunk_body, 0, unroll=8)

    def pair_body(g, _):
      r0 = 2 * g
      b0 = base + r0

      @pl.when(g > 0)
      def _():
        pltpu.make_async_copy(buf0, out_hbm.at[b0 - 2], so0).wait()

      dx0 = pltpu.async_copy(x_hbm.at[b0], buf0.at[pl.ds(0, xw)], sx0)

      @pl.when(g > 0)
      def _():
        pltpu.make_async_copy(buf1, out_hbm.at[b0 - 1], so1).wait()

      dx1 = pltpu.async_copy(x_hbm.at[b0 + 1], buf1.at[pl.ds(0, xw)], sx1)

      compute_lc(buf0, r0)
      dx0.wait()
      pltpu.async_copy(buf0, out_hbm.at[b0], so0)

      compute_lc(buf1, r0 + 1)
      dx1.wait()
      pltpu.async_copy(buf1, out_hbm.at[b0 + 1], so1)
      return 0

    lax.fori_loop(0, rows_per_w // 2, pair_body, 0)
    pltpu.make_async_copy(buf0, out_hbm.at[base + rows_per_w - 2], so0).wait()
    pltpu.make_async_copy(buf1, out_hbm.at[base + rows_per_w - 1], so1).wait()

  return k(x2, lens, r2, posf)


def _tc_mask(lens2, pad250, B, S, M):
  W = S + M
  BB = 256
  grid = (B // BB,)

  def body(len_ref, pad_ref, out_ref):
    L = len_ref[...]
    col = lax.broadcasted_iota(jnp.int32, (BB, W), 1)
    lc = jnp.where(col - S >= L, -jnp.inf, 0.0).astype(jnp.float32)
    out_ref[...] = jnp.where(col < S, pad_ref[...], lc)

  return pl.pallas_call(
      body,
      grid=grid,
      in_specs=[
          pl.BlockSpec((BB, 1), lambda i: (i, 0)),
          pl.BlockSpec((BB, W), lambda i: (i, 0)),
      ],
      out_specs=pl.BlockSpec((BB, W), lambda i: (i, 0)),
      out_shape=jax.ShapeDtypeStruct((B, W), jnp.float32),
  )(lens2, pad250)


def kernel(x, padding_mask, target_length, rev_table, pos_table):
  B, S, D = x.shape
  M = rev_table.shape[0]
  lens = target_length.astype(jnp.int32)
  rev = rev_table[::-1]
  r2 = jnp.concatenate([rev, rev], axis=0).reshape(2 * M * D)
  posf = pos_table.reshape(M * D)
  x2 = x.reshape(B, S * D)

  out2 = _sc_fused_out(x2, lens, r2, posf, B, S, D, M)
  pad250 = jnp.pad(padding_mask.astype(jnp.float32), ((0, 0), (0, M)))
  mask_out = _tc_mask(lens.reshape(B, 1), pad250, B, S, M)
  return out2.reshape(B, S + M, D), mask_out.astype(x.dtype)


# speedup vs baseline: 6.7982x; 1.8806x over previous
import functools

import jax
import jax.numpy as jnp
from jax import lax
from jax.experimental import pallas as pl
from jax.experimental.pallas import tpu as pltpu
from jax.experimental.pallas import tpu_sc as plsc

_NC = 2
_NS = 16
_L = 16


def _sc_fused_out(x2, lens, r2, posf, B, S, D, M):
  NW = _NC * _NS
  rows_per_w = B // NW
  xw = S * D
  lw = M * D
  ow = xw + lw
  n_chunks = lw // _L

  mesh = plsc.VectorSubcoreMesh(core_axis_name="c", subcore_axis_name="s")

  @functools.partial(
      pl.kernel,
      mesh=mesh,
      out_type=jax.ShapeDtypeStruct((B, ow), jnp.float32),
      scratch_types=[
          pltpu.VMEM((2 * lw,), jnp.float32),
          pltpu.VMEM((lw,), jnp.float32),
          pltpu.VMEM((rows_per_w + _L,), jnp.int32),
          pltpu.VMEM((ow,), jnp.float32),
          pltpu.VMEM((ow,), jnp.float32),
          pltpu.SemaphoreType.DMA,
          pltpu.SemaphoreType.DMA,
          pltpu.SemaphoreType.DMA,
          pltpu.SemaphoreType.DMA,
      ],
  )
  def k(x_hbm, len_hbm, r2_hbm, pos_hbm, out_hbm,
        r2_v, pos_v, len_v, buf0, buf1, sx0, sx1, so0, so1):
    wid = lax.axis_index("s") * _NC + lax.axis_index("c")
    base = wid * rows_per_w
    pltpu.sync_copy(r2_hbm, r2_v)
    pltpu.sync_copy(pos_hbm, pos_v)
    pltpu.sync_copy(len_hbm.at[pl.ds(base, rows_per_w)],
                    len_v.at[pl.ds(0, rows_per_w)])

    def compute_lc(buf, r):
      L_b = len_v[pl.ds(r, _L)][0]
      c = (M - L_b) * D

      @plsc.parallel_loop(0, lw, _L, unroll=8)
      def _(o):
        buf[pl.ds(xw + o, _L)] = r2_v[pl.ds(c + o, _L)] + pos_v[pl.ds(o, _L)]

    def pair_body(g, _):
      r0 = 2 * g
      b0 = base + r0

      @pl.when(g > 0)
      def _():
        pltpu.make_async_copy(buf0, out_hbm.at[b0 - 2], so0).wait()

      dx0 = pltpu.async_copy(x_hbm.at[b0], buf0.at[pl.ds(0, xw)], sx0)

      @pl.when(g > 0)
      def _():
        pltpu.make_async_copy(buf1, out_hbm.at[b0 - 1], so1).wait()

      dx1 = pltpu.async_copy(x_hbm.at[b0 + 1], buf1.at[pl.ds(0, xw)], sx1)

      compute_lc(buf0, r0)
      dx0.wait()
      pltpu.async_copy(buf0, out_hbm.at[b0], so0)

      compute_lc(buf1, r0 + 1)
      dx1.wait()
      pltpu.async_copy(buf1, out_hbm.at[b0 + 1], so1)
      return 0

    lax.fori_loop(0, rows_per_w // 2, pair_body, 0)
    pltpu.make_async_copy(buf0, out_hbm.at[base + rows_per_w - 2], so0).wait()
    pltpu.make_async_copy(buf1, out_hbm.at[base + rows_per_w - 1], so1).wait()

  return k(x2, lens, r2, posf)


def _tc_mask(lens2, pad250, B, S, M):
  W = S + M
  BB = 256
  grid = (B // BB,)

  def body(len_ref, pad_ref, out_ref):
    L = len_ref[...]
    col = lax.broadcasted_iota(jnp.int32, (BB, W), 1)
    lc = jnp.where(col - S >= L, -jnp.inf, 0.0).astype(jnp.float32)
    out_ref[...] = jnp.where(col < S, pad_ref[...], lc)

  return pl.pallas_call(
      body,
      grid=grid,
      in_specs=[
          pl.BlockSpec((BB, 1), lambda i: (i, 0)),
          pl.BlockSpec((BB, W), lambda i: (i, 0)),
      ],
      out_specs=pl.BlockSpec((BB, W), lambda i: (i, 0)),
      out_shape=jax.ShapeDtypeStruct((B, W), jnp.float32),
  )(lens2, pad250)


def kernel(x, padding_mask, target_length, rev_table, pos_table):
  B, S, D = x.shape
  M = rev_table.shape[0]
  lens = target_length.astype(jnp.int32)
  rev = rev_table[::-1]
  r2 = jnp.concatenate([rev, rev], axis=0).reshape(2 * M * D)
  posf = pos_table.reshape(M * D)
  x2 = x.reshape(B, S * D)

  out2 = _sc_fused_out(x2, lens, r2, posf, B, S, D, M)
  pad250 = jnp.pad(padding_mask.astype(jnp.float32), ((0, 0), (0, M)))
  mask_out = _tc_mask(lens.reshape(B, 1), pad250, B, S, M)
  return out2.reshape(B, S + M, D), mask_out.astype(x.dtype)
